# Initial kernel scaffold; baseline (speedup 1.0000x reference)
import jax
import jax.numpy as jnp
from jax.experimental import pallas as pl


def kernel(x, edge_index, W_lin, W_ih, W_hh, b_ih, b_hh):
    raise NotImplementedError("write your pallas kernel here")


# speedup vs baseline: 6.8915x; 6.8915x over previous
import functools

import jax
import jax.numpy as jnp
from jax import lax
from jax.experimental import pallas as pl
from jax.experimental.pallas import tpu as pltpu
from jax.experimental.pallas import tpu_sc as plsc

_STEPS = 3
_NC, _NS = 2, 16
_NW = _NC * _NS
_EB = 80


def _propagate_body(n_nodes, nb, src_hbm, dst_hbm, table_hbm, zeros_hbm,
                    out_hbm, src_v, dst_v, rows_v, sem, acc_sh):
    c = lax.axis_index("c")
    s = lax.axis_index("s")
    w = s * _NC + c
    rpt = n_nodes // _NS

    pltpu.sync_copy(zeros_hbm, acc_sh.at[pl.ds(s * rpt, rpt)])

    pltpu.sync_copy(src_hbm.at[w], src_v)
    pltpu.sync_copy(dst_hbm.at[w], dst_v)
    plsc.subcore_barrier()

    def body(j, _):
        pltpu.async_copy(table_hbm.at[src_v.at[j]], rows_v, sem).wait()
        pltpu.sync_copy(rows_v, acc_sh.at[dst_v.at[j]], add=True)
        return 0

    lax.fori_loop(0, nb, body, 0)
    plsc.subcore_barrier()

    pltpu.sync_copy(acc_sh.at[pl.ds(s * rpt, rpt)], out_hbm.at[c, s])


def _make_propagate(n_nodes, n_edges, channels):
    nb = n_edges // (_NW * _EB)
    rpt = n_nodes // _NS
    mesh = plsc.VectorSubcoreMesh(core_axis_name="c", subcore_axis_name="s")
    return pl.kernel(
        functools.partial(_propagate_body, n_nodes, nb),
        out_type=jax.ShapeDtypeStruct((_NC, _NS, rpt, channels), jnp.float32),
        mesh=mesh,
        scratch_types=[
            pltpu.VMEM((nb, _EB), jnp.int32),
            pltpu.VMEM((nb, _EB), jnp.int32),
            pltpu.VMEM((_EB, channels), jnp.float32),
            pltpu.SemaphoreType.DMA,
            pltpu.VMEM_SHARED((n_nodes, channels), jnp.float32),
        ],
    )


def _transform_body(x_ref, w_ref, o_ref):
    o_ref[...] = lax.dot_general(
        x_ref[...], w_ref[...], (((1,), (1,)), ((), ())),
        preferred_element_type=jnp.float32)


def _gru_body(pp_ref, h_ref, wih_ref, whh_ref, bih_ref, bhh_ref, wlin_ref,
              h_out, t_out):
    ch = h_ref.shape[-1]
    prop = pp_ref[0] + pp_ref[1]
    h = h_ref[...]
    gi = lax.dot_general(prop, wih_ref[...], (((1,), (1,)), ((), ())),
                         preferred_element_type=jnp.float32) + bih_ref[...]
    gh = lax.dot_general(h, whh_ref[...], (((1,), (1,)), ((), ())),
                         preferred_element_type=jnp.float32) + bhh_ref[...]
    r = jax.nn.sigmoid(gi[:, :ch] + gh[:, :ch])
    z = jax.nn.sigmoid(gi[:, ch:2 * ch] + gh[:, ch:2 * ch])
    n = jnp.tanh(gi[:, 2 * ch:] + r * gh[:, 2 * ch:])
    hn = (1.0 - z) * n + z * h
    h_out[...] = hn
    t_out[...] = lax.dot_general(hn, wlin_ref[...], (((1,), (1,)), ((), ())),
                                 preferred_element_type=jnp.float32)


def _transform_call(x, w_lin, blk):
    n, ch = x.shape
    grid = n // blk
    return pl.pallas_call(
        _transform_body,
        grid=(grid,),
        in_specs=[
            pl.BlockSpec((blk, ch), lambda i: (i, 0)),
            pl.BlockSpec((ch, ch), lambda i: (0, 0)),
        ],
        out_specs=pl.BlockSpec((blk, ch), lambda i: (i, 0)),
        out_shape=jax.ShapeDtypeStruct((n, ch), jnp.float32),
    )(x, w_lin)


def _gru_call(pp, h, w_ih, w_hh, b_ih, b_hh, w_lin, blk):
    n, ch = h.shape
    grid = n // blk
    full = lambda i: (0, 0)
    return pl.pallas_call(
        _gru_body,
        grid=(grid,),
        in_specs=[
            pl.BlockSpec((_NC, blk, ch), lambda i: (0, i, 0)),
            pl.BlockSpec((blk, ch), lambda i: (i, 0)),
            pl.BlockSpec((3 * ch, ch), full),
            pl.BlockSpec((3 * ch, ch), full),
            pl.BlockSpec((1, 3 * ch), full),
            pl.BlockSpec((1, 3 * ch), full),
            pl.BlockSpec((ch, ch), full),
        ],
        out_specs=[
            pl.BlockSpec((blk, ch), lambda i: (i, 0)),
            pl.BlockSpec((blk, ch), lambda i: (i, 0)),
        ],
        out_shape=[
            jax.ShapeDtypeStruct((n, ch), jnp.float32),
            jax.ShapeDtypeStruct((n, ch), jnp.float32),
        ],
    )(pp, h, w_ih, w_hh, b_ih, b_hh, w_lin)


def kernel(x, edge_index, W_lin, W_ih, W_hh, b_ih, b_hh):
    n, ch = x.shape
    n_edges = edge_index.shape[1]
    nb = n_edges // (_NW * _EB)
    rpt = n_nodes_per_tile = n // _NS

    src = edge_index[0].astype(jnp.int32).reshape(_NW, nb, _EB)
    dst = edge_index[1].astype(jnp.int32).reshape(_NW, nb, _EB)
    zeros = jnp.zeros((rpt, ch), jnp.float32)
    bih = b_ih.reshape(1, 3 * ch)
    bhh = b_hh.reshape(1, 3 * ch)

    propagate = _make_propagate(n, n_edges, ch)
    blk = 2000

    t = _transform_call(x, W_lin, blk)
    state = x
    for _ in range(_STEPS):
        partials = propagate(src, dst, t, zeros)
        pp = partials.reshape(_NC, n, ch)
        state, t = _gru_call(pp, state, W_ih, W_hh, bih, bhh, W_lin, blk)
    return state
